# baseline (speedup 1.0000x reference)
import functools

import jax
import jax.numpy as jnp
from jax import lax
from jax.experimental import pallas as pl
from jax.experimental.pallas import tpu as pltpu
from jax.experimental.pallas import tpu_sc as plsc

_L = 16


def _make_emb_builder(R, Cg, C):
    half = C // 2
    mesh = plsc.VectorSubcoreMesh(core_axis_name="c", subcore_axis_name="s")

    @functools.partial(
        pl.kernel,
        mesh=mesh,
        out_type=jax.ShapeDtypeStruct((R * Cg, C), jnp.float32),
        scratch_types=[
            pltpu.VMEM((R, half), jnp.float32),
            pltpu.VMEM((Cg, half), jnp.float32),
            pltpu.VMEM((Cg, C), jnp.float32),
        ],
    )
    def build_emb(row_hbm, col_hbm, out_hbm, row_v, col_v, blk_v):
        w = lax.axis_index("s") * 2 + lax.axis_index("c")

        @pl.when(w < R)
        def _():
            pltpu.sync_copy(row_hbm, row_v)
            pltpu.sync_copy(col_hbm, col_v)
            for j in range(half // _L):
                v = row_v[w, _L * j:_L * (j + 1)]
                for i in range(Cg):
                    blk_v[i, _L * j:_L * (j + 1)] = v
            for i in range(Cg):
                for j in range(half // _L):
                    blk_v[i, half + _L * j:half + _L * (j + 1)] = (
                        col_v[i, _L * j:_L * (j + 1)])
            pltpu.sync_copy(blk_v, out_hbm.at[pl.ds(w * Cg, Cg)])

    return build_emb


def _tc_body(x_ref, e_ref, o_ref):
    o_ref[...] = x_ref[...] + e_ref[...]


def kernel(inputs, row_emb, col_emb):
    B, R, Cg, C = inputs.shape
    emb = _make_emb_builder(R, Cg, C)(row_emb, col_emb)
    emb4 = emb.reshape(1, R, Cg, C)
    BB = 8
    out = pl.pallas_call(
        _tc_body,
        grid=(B // BB,),
        in_specs=[
            pl.BlockSpec((BB, R, Cg, C), lambda b: (b, 0, 0, 0)),
            pl.BlockSpec((1, R, Cg, C), lambda b: (0, 0, 0, 0)),
        ],
        out_specs=pl.BlockSpec((BB, R, Cg, C), lambda b: (b, 0, 0, 0)),
        out_shape=jax.ShapeDtypeStruct((B, R, Cg, C), inputs.dtype),
    )(inputs, emb4)
    return out.reshape(B, R * Cg, C)


# speedup vs baseline: 1.7714x; 1.7714x over previous
import jax
import jax.numpy as jnp
from jax.experimental import pallas as pl
from jax.experimental.pallas import tpu as pltpu


def _body(x_ref, r_ref, c_ref, o_ref):
    x = x_ref[...]
    r = r_ref[...]
    c = c_ref[...]
    half = r.shape[-1]
    o_ref[:, :, :, :half] = x[:, :, :, :half] + r[None, :, None, :]
    o_ref[:, :, :, half:] = x[:, :, :, half:] + c[None, None, :, :]


def _add_call(inputs, row_emb, col_emb, BB):
    B, R, Cg, C = inputs.shape
    return pl.pallas_call(
        _body,
        grid=(B // BB,),
        in_specs=[
            pl.BlockSpec((BB, R, Cg, C), lambda b: (b, 0, 0, 0)),
            pl.BlockSpec((R, C // 2), lambda b: (0, 0)),
            pl.BlockSpec((Cg, C // 2), lambda b: (0, 0)),
        ],
        out_specs=pl.BlockSpec((BB, R, Cg, C), lambda b: (b, 0, 0, 0)),
        out_shape=jax.ShapeDtypeStruct((B, R, Cg, C), inputs.dtype),
    )(inputs, row_emb, col_emb)


def kernel(inputs, row_emb, col_emb):
    B, R, Cg, C = inputs.shape
    out = _add_call(inputs, row_emb, col_emb, 8)
    return out.reshape(B, R * Cg, C)
